# baseline (speedup 1.0000x reference)
import jax
import jax.numpy as jnp
from jax import lax
from jax.experimental import pallas as pl
from jax.experimental.pallas import tpu as pltpu
from jax.experimental.pallas import tpu_sc as plsc

_B, _C, _T = 16, 1024, 4096
_R = _B * _C
_N = _R * _T
_NW = 32
_RPW = _R // _NW
_K = 8
_CHUNK = _K * _T
_NCHUNK = _RPW // _K
_NV = _CHUNK // 16


def _sc_body(cache_hbm, x_hbm, out_hbm, buf0, buf1, xbuf0, xbuf1,
             sem_in, sem_x, sem_out):
    wid = lax.axis_index("s") * 2 + lax.axis_index("c")
    row0 = wid * _RPW
    lanes = lax.iota(jnp.int32, 16)

    def start_in(ci, buf, xbuf):
        rbase = row0 + ci * _K
        pltpu.make_async_copy(
            cache_hbm.at[pl.ds(rbase * _T, _CHUNK)],
            buf.at[pl.ds(0, _CHUNK)],
            sem_in,
        ).start()
        pltpu.make_async_copy(
            x_hbm.at[pl.ds(rbase, _K)],
            xbuf.at[pl.ds(0, _K)],
            sem_x,
        ).start()

    def wait_in(buf, xbuf):
        pltpu.make_async_copy(
            cache_hbm.at[pl.ds(0, _CHUNK)], buf.at[pl.ds(0, _CHUNK)], sem_in
        ).wait()
        pltpu.make_async_copy(
            x_hbm.at[pl.ds(0, _K)], xbuf.at[pl.ds(0, _K)], sem_x
        ).wait()

    def do_chunk(ci, buf, xbuf, first, last):
        rbase = row0 + ci * _K
        wait_in(buf, xbuf)

        def shift_step(i, _):
            v = plsc.load_gather(buf, [lanes + (i * 16 + 1)])
            plsc.store_scatter(buf, [lanes + i * 16], v)
            return ()

        lax.fori_loop(0, _NV, shift_step, (), unroll=8)
        xv = xbuf[pl.ds(0, 16)]
        plsc.store_scatter(buf, [lanes * _T + (_T - 1)], xv, mask=lanes < _K)
        cp_out = pltpu.make_async_copy(
            buf.at[pl.ds(0, _CHUNK)],
            out_hbm.at[pl.ds(rbase * _T, _CHUNK)],
            sem_out,
        )
        cp_out.start()
        cp_out.wait()

    def step(i, _):
        ci0 = 2 * i
        do_chunk(ci0, buf0, xbuf0, False, False)
        lax.cond(
            ci0 + 2 < _NCHUNK,
            lambda: start_in(ci0 + 2, buf0, xbuf0),
            lambda: None,
        )
        do_chunk(ci0 + 1, buf1, xbuf1, False, False)
        lax.cond(
            ci0 + 3 < _NCHUNK,
            lambda: start_in(ci0 + 3, buf1, xbuf1),
            lambda: None,
        )
        return ()

    start_in(0, buf0, xbuf0)
    start_in(1, buf1, xbuf1)
    lax.fori_loop(0, _NCHUNK // 2, step, ())


def kernel(cache, x):
    cache_flat = cache.reshape(_N)
    x_flat = x.reshape(_R)
    mesh = plsc.VectorSubcoreMesh(core_axis_name="c", subcore_axis_name="s")
    out = pl.kernel(
        _sc_body,
        out_type=jax.ShapeDtypeStruct((_N,), cache.dtype),
        mesh=mesh,
        compiler_params=pltpu.CompilerParams(needs_layout_passes=False),
        scratch_types=[
            pltpu.VMEM((_CHUNK + 16,), jnp.float32),
            pltpu.VMEM((_CHUNK + 16,), jnp.float32),
            pltpu.VMEM((16,), jnp.float32),
            pltpu.VMEM((16,), jnp.float32),
            pltpu.SemaphoreType.DMA,
            pltpu.SemaphoreType.DMA,
            pltpu.SemaphoreType.DMA,
        ],
    )(cache_flat, x_flat)
    return out.reshape(_B, _C, _T)


# speedup vs baseline: 1.0842x; 1.0842x over previous
import jax
import jax.numpy as jnp
from jax import lax
from jax.experimental import pallas as pl
from jax.experimental.pallas import tpu as pltpu
from jax.experimental.pallas import tpu_sc as plsc

_B, _C, _T = 16, 1024, 4096
_R = _B * _C
_N = _R * _T
_NW = 32
_RPW = _R // _NW
_K = 4
_CHUNK = _K * _T
_NCHUNK = _RPW // _K
_NV = _CHUNK // 16


def _sc_body(cache_hbm, x_hbm, out_hbm, src0, src1, dst0, dst1, xw,
             si0, si1, so0, so1, sxw):
    wid = lax.axis_index("s") * 2 + lax.axis_index("c")
    row0 = wid * _RPW
    lanes = lax.iota(jnp.int32, 16)
    idx_rd = lanes + 1
    idx_xs = lanes * _T + (_T - 1)

    pltpu.make_async_copy(
        x_hbm.at[pl.ds(row0, _RPW)], xw.at[pl.ds(0, _RPW)], sxw
    ).start()

    def start_in(ci, src, si):
        rbase = row0 + ci * _K
        pltpu.make_async_copy(
            cache_hbm.at[pl.ds(rbase * _T, _CHUNK)],
            src.at[pl.ds(0, _CHUNK)],
            si,
        ).start()

    def wait_in(src, si):
        pltpu.make_async_copy(
            cache_hbm.at[pl.ds(0, _CHUNK)], src.at[pl.ds(0, _CHUNK)], si
        ).wait()

    def wait_out(dst, so):
        pltpu.make_async_copy(
            dst.at[pl.ds(0, _CHUNK)], out_hbm.at[pl.ds(0, _CHUNK)], so
        ).wait()

    def do_chunk(ci, src, dst, si, so, drain_prev):
        rbase = row0 + ci * _K
        wait_in(src, si)
        lax.cond(drain_prev, lambda: wait_out(dst, so), lambda: None)

        def shift_step(i, _):
            v = plsc.load_gather(src, [idx_rd + i * 16])
            plsc.store_scatter(dst, [lanes + i * 16], v)
            return ()

        lax.fori_loop(0, _NV, shift_step, (), unroll=16)
        xv = plsc.load_gather(xw, [ci * _K + lanes])
        plsc.store_scatter(dst, [idx_xs], xv, mask=lanes < _K)
        pltpu.make_async_copy(
            dst.at[pl.ds(0, _CHUNK)],
            out_hbm.at[pl.ds(rbase * _T, _CHUNK)],
            so,
        ).start()

    start_in(0, src0, si0)
    start_in(1, src1, si1)
    pltpu.make_async_copy(
        x_hbm.at[pl.ds(0, _RPW)], xw.at[pl.ds(0, _RPW)], sxw
    ).wait()

    def step(i, _):
        ci0 = 2 * i
        do_chunk(ci0, src0, dst0, si0, so0, i > 0)
        lax.cond(
            ci0 + 2 < _NCHUNK,
            lambda: start_in(ci0 + 2, src0, si0),
            lambda: None,
        )
        do_chunk(ci0 + 1, src1, dst1, si1, so1, i > 0)
        lax.cond(
            ci0 + 3 < _NCHUNK,
            lambda: start_in(ci0 + 3, src1, si1),
            lambda: None,
        )
        return ()

    lax.fori_loop(0, _NCHUNK // 2, step, ())
    wait_out(dst0, so0)
    wait_out(dst1, so1)


def kernel(cache, x):
    cache_flat = cache.reshape(_N)
    x_flat = x.reshape(_R)
    mesh = plsc.VectorSubcoreMesh(core_axis_name="c", subcore_axis_name="s")
    out = pl.kernel(
        _sc_body,
        out_type=jax.ShapeDtypeStruct((_N,), cache.dtype),
        mesh=mesh,
        compiler_params=pltpu.CompilerParams(needs_layout_passes=False),
        scratch_types=[
            pltpu.VMEM((_CHUNK + 16,), jnp.float32),
            pltpu.VMEM((_CHUNK + 16,), jnp.float32),
            pltpu.VMEM((_CHUNK,), jnp.float32),
            pltpu.VMEM((_CHUNK,), jnp.float32),
            pltpu.VMEM((_RPW + 16,), jnp.float32),
            pltpu.SemaphoreType.DMA,
            pltpu.SemaphoreType.DMA,
            pltpu.SemaphoreType.DMA,
            pltpu.SemaphoreType.DMA,
            pltpu.SemaphoreType.DMA,
        ],
    )(cache_flat, x_flat)
    return out.reshape(_B, _C, _T)


# speedup vs baseline: 1.6565x; 1.5278x over previous
import jax
import jax.numpy as jnp
from jax import lax
from jax.experimental import pallas as pl
from jax.experimental.pallas import tpu as pltpu
from jax.experimental.pallas import tpu_sc as plsc

_B, _C, _T = 16, 1024, 4096
_R = _B * _C
_N = _R * _T
_NW = 32
_RPW = _R // _NW
_K = 4
_CHUNK = _K * _T
_NCHUNK = _RPW // _K
_NV = _CHUNK // 16


def _sc_body(cache_hbm, x_hbm, out_hbm, src0, src1, dst0, dst1, xw,
             si0, si1, so0, so1, sxw):
    wid = lax.axis_index("s") * 2 + lax.axis_index("c")
    row0 = wid * _RPW
    lanes = lax.iota(jnp.int32, 16)
    idx_rd = lanes + 1
    idx_xs = lanes * _T + (_T - 1)

    pltpu.make_async_copy(
        x_hbm.at[pl.ds(row0, _RPW)], xw.at[pl.ds(0, _RPW)], sxw
    ).start()

    def start_in(ci, src, si):
        rbase = row0 + ci * _K
        pltpu.make_async_copy(
            cache_hbm.at[pl.ds(rbase * _T, _CHUNK)],
            src.at[pl.ds(0, _CHUNK)],
            si,
        ).start()

    def wait_in(src, si):
        pltpu.make_async_copy(
            cache_hbm.at[pl.ds(0, _CHUNK)], src.at[pl.ds(0, _CHUNK)], si
        ).wait()

    def wait_out(dst, so):
        pltpu.make_async_copy(
            dst.at[pl.ds(0, _CHUNK)], out_hbm.at[pl.ds(0, _CHUNK)], so
        ).wait()

    def do_chunk(ci, src, dst, si, so, drain_prev):
        rbase = row0 + ci * _K
        wait_in(src, si)
        lax.cond(drain_prev, lambda: wait_out(dst, so), lambda: None)

        @plsc.parallel_loop(0, _NV, unroll=8)
        def _shift_loop(i):
            v = plsc.load_gather(src, [idx_rd + i * 16])
            plsc.store_scatter(dst, [lanes + i * 16], v)
        xv = plsc.load_gather(xw, [ci * _K + lanes])
        plsc.store_scatter(dst, [idx_xs], xv, mask=lanes < _K)
        pltpu.make_async_copy(
            dst.at[pl.ds(0, _CHUNK)],
            out_hbm.at[pl.ds(rbase * _T, _CHUNK)],
            so,
        ).start()

    start_in(0, src0, si0)
    start_in(1, src1, si1)
    pltpu.make_async_copy(
        x_hbm.at[pl.ds(0, _RPW)], xw.at[pl.ds(0, _RPW)], sxw
    ).wait()

    def step(i, _):
        ci0 = 2 * i
        do_chunk(ci0, src0, dst0, si0, so0, i > 0)
        lax.cond(
            ci0 + 2 < _NCHUNK,
            lambda: start_in(ci0 + 2, src0, si0),
            lambda: None,
        )
        do_chunk(ci0 + 1, src1, dst1, si1, so1, i > 0)
        lax.cond(
            ci0 + 3 < _NCHUNK,
            lambda: start_in(ci0 + 3, src1, si1),
            lambda: None,
        )
        return ()

    lax.fori_loop(0, _NCHUNK // 2, step, ())
    wait_out(dst0, so0)
    wait_out(dst1, so1)


def kernel(cache, x):
    cache_flat = cache.reshape(_N)
    x_flat = x.reshape(_R)
    mesh = plsc.VectorSubcoreMesh(core_axis_name="c", subcore_axis_name="s")
    out = pl.kernel(
        _sc_body,
        out_type=jax.ShapeDtypeStruct((_N,), cache.dtype),
        mesh=mesh,
        compiler_params=pltpu.CompilerParams(needs_layout_passes=False),
        scratch_types=[
            pltpu.VMEM((_CHUNK + 16,), jnp.float32),
            pltpu.VMEM((_CHUNK + 16,), jnp.float32),
            pltpu.VMEM((_CHUNK,), jnp.float32),
            pltpu.VMEM((_CHUNK,), jnp.float32),
            pltpu.VMEM((_RPW + 16,), jnp.float32),
            pltpu.SemaphoreType.DMA,
            pltpu.SemaphoreType.DMA,
            pltpu.SemaphoreType.DMA,
            pltpu.SemaphoreType.DMA,
            pltpu.SemaphoreType.DMA,
        ],
    )(cache_flat, x_flat)
    return out.reshape(_B, _C, _T)


# speedup vs baseline: 6.1499x; 3.7125x over previous
import jax
import jax.numpy as jnp
from jax.experimental import pallas as pl
from jax.experimental.pallas import tpu as pltpu

_B, _C, _T = 16, 1024, 4096
_R = _B * _C
_ROWS_BLK = 512


def _shift_body(cache_ref, x_ref, out_ref):
    blk = cache_ref[...]
    out_ref[...] = jnp.concatenate([blk[:, 1:], x_ref[...]], axis=1)


def kernel(cache, x):
    cache2 = cache.reshape(_R, _T)
    x2 = x.reshape(_R, 1)
    out = pl.pallas_call(
        _shift_body,
        grid=(_R // _ROWS_BLK,),
        in_specs=[
            pl.BlockSpec((_ROWS_BLK, _T), lambda i: (i, 0)),
            pl.BlockSpec((_ROWS_BLK, 1), lambda i: (i, 0)),
        ],
        out_specs=pl.BlockSpec((_ROWS_BLK, _T), lambda i: (i, 0)),
        out_shape=jax.ShapeDtypeStruct((_R, _T), cache.dtype),
    )(cache2, x2)
    return out.reshape(_B, _C, _T)


# speedup vs baseline: 6.1844x; 1.0056x over previous
import jax
import jax.numpy as jnp
from jax.experimental import pallas as pl
from jax.experimental.pallas import tpu as pltpu

_B, _C, _T = 16, 1024, 4096
_R = _B * _C
_ROWS_BLK = 688


def _shift_body(cache_ref, x_ref, out_ref):
    blk = cache_ref[...]
    out_ref[...] = jnp.concatenate([blk[:, 1:], x_ref[...]], axis=1)


def kernel(cache, x):
    cache2 = cache.reshape(_R, _T)
    x2 = x.reshape(_R, 1)
    out = pl.pallas_call(
        _shift_body,
        grid=(pl.cdiv(_R, _ROWS_BLK),),
        in_specs=[
            pl.BlockSpec((_ROWS_BLK, _T), lambda i: (i, 0),
                         ),
            pl.BlockSpec((_ROWS_BLK, 1), lambda i: (i, 0),
                         ),
        ],
        out_specs=pl.BlockSpec((_ROWS_BLK, _T), lambda i: (i, 0),
                               ),
        out_shape=jax.ShapeDtypeStruct((_R, _T), cache.dtype),
    )(cache2, x2)
    return out.reshape(_B, _C, _T)
